# Initial kernel scaffold; baseline (speedup 1.0000x reference)
import jax
import jax.numpy as jnp
from jax.experimental import pallas as pl


def kernel(img, seg, fV_regions):
    raise NotImplementedError("write your pallas kernel here")


# speedup vs baseline: 21.0761x; 21.0761x over previous
import jax
import jax.numpy as jnp
from jax import lax
from jax.experimental import pallas as pl
from jax.experimental.pallas import tpu as pltpu
from jax.experimental.pallas import tpu_sc as plsc

B, C, H, W = 8, 3, 512, 512
HW = H * W
N = B * HW
NV = 131072
NW = 32
P = N // NW
M = 2048
G = M // 128
NSUB = P // M
RT = NV // 16


def _iota16():
    return lax.iota(jnp.int32, 16)


def _accum_body(img_hbm, seg_hbm, t_out, idx2d, xb0, xb1, xb2, ones, zb,
                tb0, tb1, tb2, tb3, sem):
    cid = lax.axis_index("c")
    sid = lax.axis_index("s")
    wid = sid * 2 + cid
    iota = _iota16()
    tabs = (tb0, tb1, tb2, tb3)

    def fill(i, _):
        ones[pl.ds(i * 16, 16)] = jnp.full((16,), 1.0, jnp.float32)
        zb[pl.ds(i * 16, 16)] = jnp.zeros((16,), jnp.float32)
        return 0
    lax.fori_loop(0, M // 16, fill, 0)

    def ztab(k, _):
        r0 = pl.multiple_of(sid * RT + k * M, M)
        for t in tabs:
            pltpu.sync_copy(zb, t.at[pl.ds(r0, M)])
        return 0
    lax.fori_loop(0, RT // M, ztab, 0)
    plsc.subcore_barrier()

    base_px = wid * P
    b = base_px // HW
    hw0 = base_px % HW

    def sub(j, _):
        px0 = base_px + j * M
        pltpu.sync_copy(seg_hbm.at[pl.ds(pl.multiple_of(px0 // 128, G), G)],
                        idx2d)
        for c, xb in enumerate((xb0, xb1, xb2)):
            off = b * (3 * HW) + c * HW + hw0 + j * M
            pltpu.sync_copy(img_hbm.at[pl.ds(pl.multiple_of(off, M), M)], xb)

        def scat(g, _):
            s0 = pl.multiple_of(g * 128, 128)
            idx = idx2d.at[g]
            cps = [pltpu.async_copy(src.at[pl.ds(s0, 128)], t.at[idx], sem,
                                    add=True)
                   for src, t in zip((xb0, xb1, xb2, ones), tabs)]
            for cp in cps:
                cp.wait()
            return 0
        lax.fori_loop(0, G, scat, 0)
        return 0
    lax.fori_loop(0, NSUB, sub, 0)
    plsc.subcore_barrier()

    def dump(k, _):
        r0 = pl.multiple_of(sid * RT + k * M, M)
        for c, t in enumerate(tabs):
            pltpu.sync_copy(t.at[pl.ds(r0, M)], t_out.at[cid, c, pl.ds(r0, M)])
        return 0
    lax.fori_loop(0, RT // M, dump, 0)


def _apply_body(img_hbm, seg_hbm, t_in, fvr_hbm, out_hbm, idx2d, xb0, xb1,
                xb2, a0, a1, a2, sb, cb, ab, fb, obuf, aj0, aj1, aj2, sem):
    cid = lax.axis_index("c")
    sid = lax.axis_index("s")
    wid = sid * 2 + cid
    iota = _iota16()
    ajs = (aj0, aj1, aj2)

    def pa(k, _):
        r0 = pl.multiple_of(sid * RT + k * M, M)
        pltpu.sync_copy(t_in.at[0, 3, pl.ds(r0, M)], cb)
        pltpu.sync_copy(t_in.at[1, 3, pl.ds(r0, M)], ab)

        def inv_cnt(i, _):
            s = pl.ds(i * 16, 16)
            cb[s] = 1.0 / jnp.maximum(cb[s] + ab[s], 1.0)
            return 0
        lax.fori_loop(0, M // 16, inv_cnt, 0)

        pltpu.sync_copy(fvr_hbm.at[pl.ds(pl.multiple_of(r0 * 3, M), 3 * M)],
                        fb)
        for c in range(3):
            pltpu.sync_copy(t_in.at[0, c, pl.ds(r0, M)], sb)
            pltpu.sync_copy(t_in.at[1, c, pl.ds(r0, M)], ab)

            def ca(i, _):
                s = pl.ds(i * 16, 16)
                f = plsc.load_gather(fb, [(i * 16 + iota) * 3 + c])
                ab[s] = f - (sb[s] + ab[s]) * cb[s]
                return 0
            lax.fori_loop(0, M // 16, ca, 0)
            pltpu.sync_copy(ab, ajs[c].at[pl.ds(r0, M)])
        return 0
    lax.fori_loop(0, RT // M, pa, 0)
    plsc.subcore_barrier()

    base_px = wid * P
    b = base_px // HW
    hw0 = base_px % HW

    def pb(j, _):
        px0 = base_px + j * M
        pltpu.sync_copy(seg_hbm.at[pl.ds(pl.multiple_of(px0 // 128, G), G)],
                        idx2d)
        for c, xb in enumerate((xb0, xb1, xb2)):
            off = b * (3 * HW) + c * HW + hw0 + j * M
            pltpu.sync_copy(img_hbm.at[pl.ds(pl.multiple_of(off, M), M)], xb)

        def gath(g, _):
            s0 = pl.multiple_of(g * 128, 128)
            idx = idx2d.at[g]
            cps = [pltpu.async_copy(t.at[idx], dst.at[pl.ds(s0, 128)], sem)
                   for t, dst in zip(ajs, (a0, a1, a2))]
            for cp in cps:
                cp.wait()
            return 0
        lax.fori_loop(0, G, gath, 0)

        def cbod(i, _):
            r = i * 16 + iota
            s = pl.ds(i * 16, 16)
            for c, (xb, a) in enumerate(zip((xb0, xb1, xb2), (a0, a1, a2))):
                plsc.store_scatter(obuf, [r * 3 + c], xb[s] + a[s])
            return 0
        lax.fori_loop(0, M // 16, cbod, 0)
        pltpu.sync_copy(obuf, out_hbm.at[pl.ds(pl.multiple_of(px0 * 3, M),
                                               3 * M)])
        return 0
    lax.fori_loop(0, NSUB, pb, 0)


def kernel(img, seg, fV_regions):
    img_flat = img.reshape(-1)
    seg2d = seg.reshape(-1, 128)
    fvr_flat = fV_regions.reshape(-1)
    mesh = plsc.VectorSubcoreMesh(core_axis_name="c", subcore_axis_name="s")

    accum = pl.kernel(
        _accum_body,
        out_type=jax.ShapeDtypeStruct((2, 4, NV), jnp.float32),
        mesh=mesh,
        compiler_params=pltpu.CompilerParams(needs_layout_passes=False),
        scratch_types=[
            pltpu.VMEM((G, 128), jnp.int32),
            pltpu.VMEM((M,), jnp.float32),
            pltpu.VMEM((M,), jnp.float32),
            pltpu.VMEM((M,), jnp.float32),
            pltpu.VMEM((M,), jnp.float32),
            pltpu.VMEM((M,), jnp.float32),
            pltpu.VMEM_SHARED((NV,), jnp.float32),
            pltpu.VMEM_SHARED((NV,), jnp.float32),
            pltpu.VMEM_SHARED((NV,), jnp.float32),
            pltpu.VMEM_SHARED((NV,), jnp.float32),
            pltpu.SemaphoreType.DMA,
        ],
    )
    t_part = accum(img_flat, seg2d)

    apply_k = pl.kernel(
        _apply_body,
        out_type=jax.ShapeDtypeStruct((N * 3,), jnp.float32),
        mesh=mesh,
        compiler_params=pltpu.CompilerParams(needs_layout_passes=False),
        scratch_types=[
            pltpu.VMEM((G, 128), jnp.int32),
            pltpu.VMEM((M,), jnp.float32),
            pltpu.VMEM((M,), jnp.float32),
            pltpu.VMEM((M,), jnp.float32),
            pltpu.VMEM((M,), jnp.float32),
            pltpu.VMEM((M,), jnp.float32),
            pltpu.VMEM((M,), jnp.float32),
            pltpu.VMEM((M,), jnp.float32),
            pltpu.VMEM((M,), jnp.float32),
            pltpu.VMEM((M,), jnp.float32),
            pltpu.VMEM((3 * M,), jnp.float32),
            pltpu.VMEM((3 * M,), jnp.float32),
            pltpu.VMEM_SHARED((NV,), jnp.float32),
            pltpu.VMEM_SHARED((NV,), jnp.float32),
            pltpu.VMEM_SHARED((NV,), jnp.float32),
            pltpu.SemaphoreType.DMA,
        ],
    )
    out = apply_k(img_flat, seg2d, t_part, fvr_flat)
    return out.reshape(N, 3)


# speedup vs baseline: 21.0815x; 1.0003x over previous
import jax
import jax.numpy as jnp
from jax import lax
from jax.experimental import pallas as pl
from jax.experimental.pallas import tpu as pltpu
from jax.experimental.pallas import tpu_sc as plsc

B, C, H, W = 8, 3, 512, 512
HW = H * W
N = B * HW
NV = 131072
NW = 32
P = N // NW
M = 2048
G = M // 128
NSUB = P // M
RT = NV // 16


def _iota16():
    return lax.iota(jnp.int32, 16)


def _accum_body(img_hbm, seg_hbm, t_out, idx2d, xb0, xb1, xb2, ones, zb,
                tb0, tb1, tb2, tb3, sem):
    cid = lax.axis_index("c")
    sid = lax.axis_index("s")
    wid = sid * 2 + cid
    tabs = (tb0, tb1, tb2, tb3)

    def fill(i, _):
        ones[pl.ds(i * 16, 16)] = jnp.full((16,), 1.0, jnp.float32)
        zb[pl.ds(i * 16, 16)] = jnp.zeros((16,), jnp.float32)
        return 0
    lax.fori_loop(0, M // 16, fill, 0)

    def ztab(k, _):
        r0 = pl.multiple_of(sid * RT + k * M, M)
        for t in tabs:
            pltpu.sync_copy(zb, t.at[pl.ds(r0, M)])
        return 0
    lax.fori_loop(0, RT // M, ztab, 0)
    plsc.subcore_barrier()

    base_px = wid * P
    b = base_px // HW
    hw0 = base_px % HW

    def sub(j, _):
        px0 = base_px + j * M
        pltpu.sync_copy(seg_hbm.at[pl.ds(pl.multiple_of(px0 // 128, G), G)],
                        idx2d)
        for c, xb in enumerate((xb0, xb1, xb2)):
            off = b * (3 * HW) + c * HW + hw0 + j * M
            pltpu.sync_copy(img_hbm.at[pl.ds(pl.multiple_of(off, M), M)], xb)

        def scat(g, _):
            s0 = pl.multiple_of(g * 128, 128)
            idx = idx2d.at[g]
            cps = [pltpu.async_copy(src.at[pl.ds(s0, 128)], t.at[idx], sem,
                                    add=True)
                   for src, t in zip((xb0, xb1, xb2, ones), tabs)]
            for cp in cps:
                cp.wait()
            return 0
        lax.fori_loop(0, G, scat, 0)
        return 0
    lax.fori_loop(0, NSUB, sub, 0)
    plsc.subcore_barrier()

    def dump(k, _):
        r0 = pl.multiple_of(sid * RT + k * M, M)
        for c, t in enumerate(tabs):
            pltpu.sync_copy(t.at[pl.ds(r0, M)], t_out.at[cid, c, pl.ds(r0, M)])
        return 0
    lax.fori_loop(0, RT // M, dump, 0)


def _apply_body(img_hbm, seg_hbm, t_in, fvr_hbm, out_hbm, idx2d, xb0, xb1,
                xb2, a0, a1, a2, sb, cb, ab, fb, obuf, aj0, aj1, aj2, sem):
    cid = lax.axis_index("c")
    sid = lax.axis_index("s")
    wid = sid * 2 + cid
    iota = _iota16()
    ajs = (aj0, aj1, aj2)

    def pa(k, _):
        r0 = pl.multiple_of(sid * RT + k * M, M)
        pltpu.sync_copy(t_in.at[0, 3, pl.ds(r0, M)], cb)
        pltpu.sync_copy(t_in.at[1, 3, pl.ds(r0, M)], ab)

        def inv_cnt(i, _):
            s = pl.ds(i * 16, 16)
            cb[s] = 1.0 / jnp.maximum(cb[s] + ab[s], 1.0)
            return 0
        lax.fori_loop(0, M // 16, inv_cnt, 0)

        pltpu.sync_copy(fvr_hbm.at[pl.ds(pl.multiple_of(r0 * 3, M), 3 * M)],
                        fb)
        for c in range(3):
            pltpu.sync_copy(t_in.at[0, c, pl.ds(r0, M)], sb)
            pltpu.sync_copy(t_in.at[1, c, pl.ds(r0, M)], ab)

            def ca(i, _):
                s = pl.ds(i * 16, 16)
                f = plsc.load_gather(fb, [(i * 16 + iota) * 3 + c])
                ab[s] = f - (sb[s] + ab[s]) * cb[s]
                return 0
            lax.fori_loop(0, M // 16, ca, 0)
            pltpu.sync_copy(ab, ajs[c].at[pl.ds(r0, M)])
        return 0
    lax.fori_loop(0, RT // M, pa, 0)
    plsc.subcore_barrier()

    base_px = wid * P
    b = base_px // HW
    hw0 = base_px % HW

    def pb(j, _):
        px0 = base_px + j * M
        pltpu.sync_copy(seg_hbm.at[pl.ds(pl.multiple_of(px0 // 128, G), G)],
                        idx2d)
        for c, xb in enumerate((xb0, xb1, xb2)):
            off = b * (3 * HW) + c * HW + hw0 + j * M
            pltpu.sync_copy(img_hbm.at[pl.ds(pl.multiple_of(off, M), M)], xb)

        def gath(g, _):
            s0 = pl.multiple_of(g * 128, 128)
            idx = idx2d.at[g]
            cps = [pltpu.async_copy(t.at[idx], dst.at[pl.ds(s0, 128)], sem)
                   for t, dst in zip(ajs, (a0, a1, a2))]
            for cp in cps:
                cp.wait()
            return 0
        lax.fori_loop(0, G, gath, 0)

        def cbod(i, _):
            r = i * 16 + iota
            s = pl.ds(i * 16, 16)
            for c, (xb, a) in enumerate(zip((xb0, xb1, xb2), (a0, a1, a2))):
                plsc.store_scatter(obuf, [r * 3 + c], xb[s] + a[s])
            return 0
        lax.fori_loop(0, M // 16, cbod, 0)
        pltpu.sync_copy(obuf, out_hbm.at[pl.ds(pl.multiple_of(px0 * 3, M),
                                               3 * M)])
        return 0
    lax.fori_loop(0, NSUB, pb, 0)


def kernel(img, seg, fV_regions):
    img_flat = img.reshape(-1)
    seg2d = seg.reshape(-1, 128)
    fvr_flat = fV_regions.reshape(-1)
    mesh = plsc.VectorSubcoreMesh(core_axis_name="c", subcore_axis_name="s")

    accum = pl.kernel(
        _accum_body,
        out_type=jax.ShapeDtypeStruct((2, 4, NV), jnp.float32),
        mesh=mesh,
        compiler_params=pltpu.CompilerParams(needs_layout_passes=False),
        scratch_types=[
            pltpu.VMEM((G, 128), jnp.int32),
            pltpu.VMEM((M,), jnp.float32),
            pltpu.VMEM((M,), jnp.float32),
            pltpu.VMEM((M,), jnp.float32),
            pltpu.VMEM((M,), jnp.float32),
            pltpu.VMEM((M,), jnp.float32),
            pltpu.VMEM_SHARED((NV,), jnp.float32),
            pltpu.VMEM_SHARED((NV,), jnp.float32),
            pltpu.VMEM_SHARED((NV,), jnp.float32),
            pltpu.VMEM_SHARED((NV,), jnp.float32),
            pltpu.SemaphoreType.DMA,
        ],
    )
    t_part = accum(img_flat, seg2d)

    apply_k = pl.kernel(
        _apply_body,
        out_type=jax.ShapeDtypeStruct((N * 3,), jnp.float32),
        mesh=mesh,
        compiler_params=pltpu.CompilerParams(needs_layout_passes=False),
        scratch_types=[
            pltpu.VMEM((G, 128), jnp.int32),
            pltpu.VMEM((M,), jnp.float32),
            pltpu.VMEM((M,), jnp.float32),
            pltpu.VMEM((M,), jnp.float32),
            pltpu.VMEM((M,), jnp.float32),
            pltpu.VMEM((M,), jnp.float32),
            pltpu.VMEM((M,), jnp.float32),
            pltpu.VMEM((M,), jnp.float32),
            pltpu.VMEM((M,), jnp.float32),
            pltpu.VMEM((M,), jnp.float32),
            pltpu.VMEM((3 * M,), jnp.float32),
            pltpu.VMEM((3 * M,), jnp.float32),
            pltpu.VMEM_SHARED((NV,), jnp.float32),
            pltpu.VMEM_SHARED((NV,), jnp.float32),
            pltpu.VMEM_SHARED((NV,), jnp.float32),
            pltpu.SemaphoreType.DMA,
        ],
    )
    out = apply_k(img_flat, seg2d, t_part, fvr_flat)
    return out.reshape(N, 3)


# speedup vs baseline: 22.3362x; 1.0595x over previous
import jax
import jax.numpy as jnp
from jax import lax
from jax.experimental import pallas as pl
from jax.experimental.pallas import tpu as pltpu
from jax.experimental.pallas import tpu_sc as plsc

B, C, H, W = 8, 3, 512, 512
HW = H * W
N = B * HW
NV = 131072
NW = 32
P = N // NW
M = 4096
G = M // 128
NSUB = P // M
RT = NV // 16
MA = 2048


def _iota16():
    return lax.iota(jnp.int32, 16)


def _x_off(wid, c, j):
    b = wid // 4
    ht = (wid % 4) * 16 + j
    return ((b * C + c) * 64 + ht) * 4096


def _accum_body(img_hbm, seg_hbm, t_out, idx2d, xb0, xb1, xb2, ones, zb,
                tb0, tb1, tb2, tb3, sem):
    cid = lax.axis_index("c")
    sid = lax.axis_index("s")
    wid = sid * 2 + cid
    tabs = (tb0, tb1, tb2, tb3)

    def fill(i, _):
        ones[pl.ds(i * 16, 16)] = jnp.full((16,), 1.0, jnp.float32)
        zb[pl.ds(i * 16, 16)] = jnp.zeros((16,), jnp.float32)
        return 0
    lax.fori_loop(0, M // 16, fill, 0)

    def ztab(k, _):
        r0 = pl.multiple_of(sid * RT + k * M, M)
        for t in tabs:
            pltpu.sync_copy(zb.at[pl.ds(0, 2048)], t.at[pl.ds(r0, 2048)])
            pltpu.sync_copy(zb.at[pl.ds(2048, 2048)],
                            t.at[pl.ds(r0 + 2048, 2048)])
        return 0
    lax.fori_loop(0, RT // M, ztab, 0)
    plsc.subcore_barrier()

    base_px = wid * P

    def sub(j, _):
        px0 = base_px + j * M
        pltpu.sync_copy(seg_hbm.at[pl.ds(pl.multiple_of(px0 // 128, G), G)],
                        idx2d)
        for c, xb in enumerate((xb0, xb1, xb2)):
            off = pl.multiple_of(_x_off(wid, c, j), M)
            pltpu.sync_copy(img_hbm.at[pl.ds(off, M)], xb)

        def scat(g, _):
            rs = (g & 3) * 8 + (g >> 2)
            s0 = pl.multiple_of(rs * 128, 128)
            idx = idx2d.at[g]
            cps = [pltpu.async_copy(src.at[pl.ds(s0, 128)], t.at[idx], sem,
                                    add=True)
                   for src, t in zip((xb0, xb1, xb2), tabs)]
            cps.append(pltpu.async_copy(ones.at[pl.ds(0, 128)],
                                        tabs[3].at[idx], sem, add=True))
            for cp in cps:
                cp.wait()
            return 0
        lax.fori_loop(0, G, scat, 0)
        return 0
    lax.fori_loop(0, NSUB, sub, 0)
    plsc.subcore_barrier()

    def dump(k, _):
        r0 = pl.multiple_of(sid * RT + k * M, M)
        for c, t in enumerate(tabs):
            pltpu.sync_copy(t.at[pl.ds(r0, M)], t_out.at[cid, c, pl.ds(r0, M)])
        return 0
    lax.fori_loop(0, RT // M, dump, 0)


def _apply_body(img_hbm, seg_hbm, t_in, fvr_hbm, out_hbm, idx2d, xb0, xb1,
                xb2, a0, a1, a2, sb, cb, ab, fb, obuf, aj0, aj1, aj2, sem):
    cid = lax.axis_index("c")
    sid = lax.axis_index("s")
    wid = sid * 2 + cid
    iota = _iota16()
    ajs = (aj0, aj1, aj2)

    def pa(k, _):
        r0 = pl.multiple_of(sid * RT + k * MA, MA)
        pltpu.sync_copy(t_in.at[0, 3, pl.ds(r0, MA)], cb)
        pltpu.sync_copy(t_in.at[1, 3, pl.ds(r0, MA)], ab)

        def inv_cnt(i, _):
            s = pl.ds(i * 16, 16)
            cb[s] = 1.0 / jnp.maximum(cb[s] + ab[s], 1.0)
            return 0
        lax.fori_loop(0, MA // 16, inv_cnt, 0)

        pltpu.sync_copy(fvr_hbm.at[pl.ds(pl.multiple_of(r0 * 4, MA), 4 * MA)],
                        fb)
        for c in range(3):
            pltpu.sync_copy(t_in.at[0, c, pl.ds(r0, MA)], sb)
            pltpu.sync_copy(t_in.at[1, c, pl.ds(r0, MA)], ab)

            def ca(i, _):
                s = pl.ds(i * 16, 16)
                foff = ((i >> 3) * 4 + c) * 128 + (i & 7) * 16
                f = fb[pl.ds(foff, 16)]
                ab[s] = f - (sb[s] + ab[s]) * cb[s]
                return 0
            lax.fori_loop(0, MA // 16, ca, 0)
            pltpu.sync_copy(ab, ajs[c].at[pl.ds(r0, MA)])
        return 0
    lax.fori_loop(0, RT // MA, pa, 0)
    plsc.subcore_barrier()

    base_px = wid * P

    def pb(j, _):
        px0 = base_px + j * M
        pltpu.sync_copy(seg_hbm.at[pl.ds(pl.multiple_of(px0 // 128, G), G)],
                        idx2d)
        for c, xb in enumerate((xb0, xb1, xb2)):
            off = pl.multiple_of(_x_off(wid, c, j), M)
            pltpu.sync_copy(img_hbm.at[pl.ds(off, M)], xb)

        def gath(g, _):
            s0 = pl.multiple_of(g * 128, 128)
            idx = idx2d.at[g]
            cps = [pltpu.async_copy(t.at[idx], dst.at[pl.ds(s0, 128)], sem)
                   for t, dst in zip(ajs, (a0, a1, a2))]
            for cp in cps:
                cp.wait()
            return 0
        lax.fori_loop(0, G, gath, 0)

        def cbod(i, _):
            g = i >> 3
            rs = (g & 3) * 8 + (g >> 2)
            xoff = rs * 128 + (i & 7) * 16
            r = i * 16 + iota
            s = pl.ds(i * 16, 16)
            for c, (xb, a) in enumerate(zip((xb0, xb1, xb2), (a0, a1, a2))):
                plsc.store_scatter(obuf, [r * 3 + c],
                                   xb[pl.ds(xoff, 16)] + a[s])
            return 0
        lax.fori_loop(0, M // 16, cbod, 0)
        pltpu.sync_copy(obuf, out_hbm.at[pl.ds(pl.multiple_of(px0 * 3, M),
                                               3 * M)])
        return 0
    lax.fori_loop(0, NSUB, pb, 0)


def kernel(img, seg, fV_regions):
    img_stripe = (img.reshape(B, C, H // 8, 8, W // 128, 128)
                  .transpose(0, 1, 2, 4, 3, 5).reshape(-1))
    seg2d = seg.reshape(-1, 128)
    fvr_g4 = (jnp.pad(fV_regions, ((0, 0), (0, 1)))
              .reshape(NV // 128, 128, 4).transpose(0, 2, 1)
              .reshape(NV // 128 * 4, 128).reshape(-1))
    mesh = plsc.VectorSubcoreMesh(core_axis_name="c", subcore_axis_name="s")

    accum = pl.kernel(
        _accum_body,
        out_type=jax.ShapeDtypeStruct((2, 4, NV), jnp.float32),
        mesh=mesh,
        compiler_params=pltpu.CompilerParams(needs_layout_passes=False),
        scratch_types=[
            pltpu.VMEM((G, 128), jnp.int32),
            pltpu.VMEM((M,), jnp.float32),
            pltpu.VMEM((M,), jnp.float32),
            pltpu.VMEM((M,), jnp.float32),
            pltpu.VMEM((M,), jnp.float32),
            pltpu.VMEM((M,), jnp.float32),
            pltpu.VMEM_SHARED((NV,), jnp.float32),
            pltpu.VMEM_SHARED((NV,), jnp.float32),
            pltpu.VMEM_SHARED((NV,), jnp.float32),
            pltpu.VMEM_SHARED((NV,), jnp.float32),
            pltpu.SemaphoreType.DMA,
        ],
    )
    t_part = accum(img_stripe, seg2d)

    apply_k = pl.kernel(
        _apply_body,
        out_type=jax.ShapeDtypeStruct((N * 3,), jnp.float32),
        mesh=mesh,
        compiler_params=pltpu.CompilerParams(needs_layout_passes=False),
        scratch_types=[
            pltpu.VMEM((G, 128), jnp.int32),
            pltpu.VMEM((M,), jnp.float32),
            pltpu.VMEM((M,), jnp.float32),
            pltpu.VMEM((M,), jnp.float32),
            pltpu.VMEM((M,), jnp.float32),
            pltpu.VMEM((M,), jnp.float32),
            pltpu.VMEM((M,), jnp.float32),
            pltpu.VMEM((MA,), jnp.float32),
            pltpu.VMEM((MA,), jnp.float32),
            pltpu.VMEM((MA,), jnp.float32),
            pltpu.VMEM((4 * MA,), jnp.float32),
            pltpu.VMEM((3 * M,), jnp.float32),
            pltpu.VMEM_SHARED((NV,), jnp.float32),
            pltpu.VMEM_SHARED((NV,), jnp.float32),
            pltpu.VMEM_SHARED((NV,), jnp.float32),
            pltpu.SemaphoreType.DMA,
        ],
    )
    out = apply_k(img_stripe, seg2d, t_part, fvr_g4)
    return out.reshape(N, 3)


# speedup vs baseline: 86.4044x; 3.8684x over previous
import jax
import jax.numpy as jnp
from jax import lax
from jax.experimental import pallas as pl
from jax.experimental.pallas import tpu as pltpu
from jax.experimental.pallas import tpu_sc as plsc

B, C, H, W = 8, 3, 512, 512
HW = H * W
N = B * HW
NV = 131072
NW = 32
P = N // NW
M = 4096
G = M // 128
NSUB = P // M
RT = NV // 16
MA = 2048


def _iota16():
    return lax.iota(jnp.int32, 16)


def _x_off(wid, c, j):
    b = wid // 4
    ht = (wid % 4) * 16 + j
    return ((b * C + c) * 64 + ht) * 4096


def _accum_body(img_hbm, seg_hbm, t_out, idx2d, xb0, xb1, xb2, ones, zb,
                tb0, tb1, tb2, tb3, sem):
    cid = lax.axis_index("c")
    sid = lax.axis_index("s")
    wid = sid * 2 + cid
    tabs = (tb0, tb1, tb2, tb3)

    def fill(i, _):
        ones[pl.ds(i * 16, 16)] = jnp.full((16,), 1.0, jnp.float32)
        zb[pl.ds(i * 16, 16)] = jnp.zeros((16,), jnp.float32)
        return 0
    lax.fori_loop(0, M // 16, fill, 0)

    def ztab(k, _):
        r0 = pl.multiple_of(sid * RT + k * M, M)
        for t in tabs:
            pltpu.sync_copy(zb.at[pl.ds(0, 2048)], t.at[pl.ds(r0, 2048)])
            pltpu.sync_copy(zb.at[pl.ds(2048, 2048)],
                            t.at[pl.ds(r0 + 2048, 2048)])
        return 0
    lax.fori_loop(0, RT // M, ztab, 0)
    plsc.subcore_barrier()

    base_px = wid * P

    def sub(j, _):
        px0 = base_px + j * M
        pltpu.sync_copy(seg_hbm.at[pl.ds(pl.multiple_of(px0 // 128, G), G)],
                        idx2d)
        for c, xb in enumerate((xb0, xb1, xb2)):
            off = pl.multiple_of(_x_off(wid, c, j), M)
            pltpu.sync_copy(img_hbm.at[pl.ds(off, M)], xb)

        def scat(g, _):
            rs = (g & 3) * 8 + (g >> 2)
            s0 = pl.multiple_of(rs * 128, 128)
            idx = idx2d.at[g]
            cps = [pltpu.async_copy(src.at[pl.ds(s0, 128)], t.at[idx], sem,
                                    add=True)
                   for src, t in zip((xb0, xb1, xb2), tabs)]
            cps.append(pltpu.async_copy(ones.at[pl.ds(0, 128)],
                                        tabs[3].at[idx], sem, add=True))
            for cp in cps:
                cp.wait()
            return 0
        lax.fori_loop(0, G, scat, 0)
        return 0
    lax.fori_loop(0, NSUB, sub, 0)
    plsc.subcore_barrier()

    def dump(k, _):
        r0 = pl.multiple_of(sid * RT + k * M, M)
        for c, t in enumerate(tabs):
            pltpu.sync_copy(t.at[pl.ds(r0, M)], t_out.at[cid, c, pl.ds(r0, M)])
        return 0
    lax.fori_loop(0, RT // M, dump, 0)


def _apply_body(img_hbm, seg_hbm, t_in, fvr_hbm, out_hbm, idx2d, xb0, xb1,
                xb2, a0, a1, a2, sb, cb, ab, fb, obuf, aj0, aj1, aj2, sem):
    cid = lax.axis_index("c")
    sid = lax.axis_index("s")
    wid = sid * 2 + cid
    iota = _iota16()
    ajs = (aj0, aj1, aj2)

    def pa(k, _):
        r0 = pl.multiple_of(sid * RT + k * MA, MA)
        pltpu.sync_copy(t_in.at[0, 3, pl.ds(r0, MA)], cb)
        pltpu.sync_copy(t_in.at[1, 3, pl.ds(r0, MA)], ab)

        def inv_cnt(i, _):
            s = pl.ds(i * 16, 16)
            cb[s] = 1.0 / jnp.maximum(cb[s] + ab[s], 1.0)
            return 0
        lax.fori_loop(0, MA // 16, inv_cnt, 0)

        pltpu.sync_copy(fvr_hbm.at[pl.ds(pl.multiple_of(r0 * 4, MA), 4 * MA)],
                        fb)
        for c in range(3):
            pltpu.sync_copy(t_in.at[0, c, pl.ds(r0, MA)], sb)
            pltpu.sync_copy(t_in.at[1, c, pl.ds(r0, MA)], ab)

            def ca(i, _):
                s = pl.ds(i * 16, 16)
                foff = ((i >> 3) * 4 + c) * 128 + (i & 7) * 16
                f = fb[pl.ds(foff, 16)]
                ab[s] = f - (sb[s] + ab[s]) * cb[s]
                return 0
            lax.fori_loop(0, MA // 16, ca, 0)
            pltpu.sync_copy(ab, ajs[c].at[pl.ds(r0, MA)])
        return 0
    lax.fori_loop(0, RT // MA, pa, 0)
    plsc.subcore_barrier()

    base_px = wid * P

    def pb(j, _):
        px0 = base_px + j * M
        pltpu.sync_copy(seg_hbm.at[pl.ds(pl.multiple_of(px0 // 128, G), G)],
                        idx2d)
        for c, xb in enumerate((xb0, xb1, xb2)):
            off = pl.multiple_of(_x_off(wid, c, j), M)
            pltpu.sync_copy(img_hbm.at[pl.ds(off, M)], xb)

        def gath(g, _):
            s0 = pl.multiple_of(g * 128, 128)
            idx = idx2d.at[g]
            cps = [pltpu.async_copy(t.at[idx], dst.at[pl.ds(s0, 128)], sem)
                   for t, dst in zip(ajs, (a0, a1, a2))]
            for cp in cps:
                cp.wait()
            return 0
        lax.fori_loop(0, G, gath, 0)

        def cbod(i, _):
            g = i >> 3
            rs = (g & 3) * 8 + (g >> 2)
            xoff = rs * 128 + (i & 7) * 16
            s = pl.ds(i * 16, 16)
            for c, (xb, a) in enumerate(zip((xb0, xb1, xb2), (a0, a1, a2))):
                ooff = (g * 4 + c) * 128 + (i & 7) * 16
                obuf[pl.ds(ooff, 16)] = xb[pl.ds(xoff, 16)] + a[s]
            return 0
        lax.fori_loop(0, M // 16, cbod, 0)
        pltpu.sync_copy(obuf, out_hbm.at[pl.ds(pl.multiple_of(px0 * 4, M),
                                               4 * M)])
        return 0
    lax.fori_loop(0, NSUB, pb, 0)


def kernel(img, seg, fV_regions):
    img_stripe = (img.reshape(B, C, H // 8, 8, W // 128, 128)
                  .transpose(0, 1, 2, 4, 3, 5).reshape(-1))
    seg2d = seg.reshape(-1, 128)
    fvr_g4 = (jnp.pad(fV_regions, ((0, 0), (0, 1)))
              .reshape(NV // 128, 128, 4).transpose(0, 2, 1)
              .reshape(NV // 128 * 4, 128).reshape(-1))
    mesh = plsc.VectorSubcoreMesh(core_axis_name="c", subcore_axis_name="s")

    accum = pl.kernel(
        _accum_body,
        out_type=jax.ShapeDtypeStruct((2, 4, NV), jnp.float32),
        mesh=mesh,
        compiler_params=pltpu.CompilerParams(needs_layout_passes=False),
        scratch_types=[
            pltpu.VMEM((G, 128), jnp.int32),
            pltpu.VMEM((M,), jnp.float32),
            pltpu.VMEM((M,), jnp.float32),
            pltpu.VMEM((M,), jnp.float32),
            pltpu.VMEM((M,), jnp.float32),
            pltpu.VMEM((M,), jnp.float32),
            pltpu.VMEM_SHARED((NV,), jnp.float32),
            pltpu.VMEM_SHARED((NV,), jnp.float32),
            pltpu.VMEM_SHARED((NV,), jnp.float32),
            pltpu.VMEM_SHARED((NV,), jnp.float32),
            pltpu.SemaphoreType.DMA,
        ],
    )
    t_part = accum(img_stripe, seg2d)

    apply_k = pl.kernel(
        _apply_body,
        out_type=jax.ShapeDtypeStruct((N * 4,), jnp.float32),
        mesh=mesh,
        compiler_params=pltpu.CompilerParams(needs_layout_passes=False),
        scratch_types=[
            pltpu.VMEM((G, 128), jnp.int32),
            pltpu.VMEM((M,), jnp.float32),
            pltpu.VMEM((M,), jnp.float32),
            pltpu.VMEM((M,), jnp.float32),
            pltpu.VMEM((M,), jnp.float32),
            pltpu.VMEM((M,), jnp.float32),
            pltpu.VMEM((M,), jnp.float32),
            pltpu.VMEM((MA,), jnp.float32),
            pltpu.VMEM((MA,), jnp.float32),
            pltpu.VMEM((MA,), jnp.float32),
            pltpu.VMEM((4 * MA,), jnp.float32),
            pltpu.VMEM((4 * M,), jnp.float32),
            pltpu.VMEM_SHARED((NV,), jnp.float32),
            pltpu.VMEM_SHARED((NV,), jnp.float32),
            pltpu.VMEM_SHARED((NV,), jnp.float32),
            pltpu.SemaphoreType.DMA,
        ],
    )
    out = apply_k(img_stripe, seg2d, t_part, fvr_g4)
    return (out.reshape(N // 128, 4, 128).transpose(0, 2, 1)
            .reshape(N, 4)[:, :3])


# speedup vs baseline: 101.6330x; 1.1762x over previous
import jax
import jax.numpy as jnp
from jax import lax
from jax.experimental import pallas as pl
from jax.experimental.pallas import tpu as pltpu
from jax.experimental.pallas import tpu_sc as plsc

B, C, H, W = 8, 3, 512, 512
HW = H * W
N = B * HW
NV = 131072
NW = 32
P = N // NW
M = 8192
G = M // 128
NSUB = P // M
RT = NV // 16
MA = 2048


def _iota16():
    return lax.iota(jnp.int32, 16)


def _x_off(wid, c, j):
    b = wid // 4
    ht = (wid % 4) * 16 + j * 2
    return ((b * C + c) * 64 + ht) * 4096


def _accum_body(img_hbm, seg_hbm, t_out, idx2d, xb0, xb1, xb2, ones, zb,
                tb0, tb1, tb2, tb3, sem, sem2):
    cid = lax.axis_index("c")
    sid = lax.axis_index("s")
    wid = sid * 2 + cid
    tabs = (tb0, tb1, tb2, tb3)

    def fill(i, _):
        ones[pl.ds(i * 16, 16)] = jnp.full((16,), 1.0, jnp.float32)
        zb[pl.ds(i * 16, 16)] = jnp.zeros((16,), jnp.float32)
        return 0
    lax.fori_loop(0, M // 16, fill, 0)

    def ztab(k, _):
        r0 = pl.multiple_of(sid * RT + k * 2048, 2048)
        for t in tabs:
            pltpu.sync_copy(zb.at[pl.ds(0, 2048)], t.at[pl.ds(r0, 2048)])
        return 0
    lax.fori_loop(0, RT // 2048, ztab, 0)
    plsc.subcore_barrier()

    base_px = wid * P

    def sub(j, _):
        px0 = base_px + j * M
        lds = [pltpu.async_copy(
            seg_hbm.at[pl.ds(pl.multiple_of(px0 // 128, G), G)], idx2d,
            sem2)]
        for c, xb in enumerate((xb0, xb1, xb2)):
            off = pl.multiple_of(_x_off(wid, c, j), M)
            lds.append(pltpu.async_copy(img_hbm.at[pl.ds(off, M)], xb, sem2))
        for ld in lds:
            ld.wait()

        def scat(g, _):
            rs = (g >> 5) * 32 + (g & 3) * 8 + ((g >> 2) & 7)
            s0 = pl.multiple_of(rs * 128, 128)
            idx = idx2d.at[g]
            cps = [pltpu.async_copy(src.at[pl.ds(s0, 128)], t.at[idx], sem,
                                    add=True)
                   for src, t in zip((xb0, xb1, xb2), tabs)]
            cps.append(pltpu.async_copy(ones.at[pl.ds(0, 128)],
                                        tabs[3].at[idx], sem, add=True))
            for cp in cps:
                cp.wait()
            return 0
        lax.fori_loop(0, G, scat, 0)
        return 0
    lax.fori_loop(0, NSUB, sub, 0)
    plsc.subcore_barrier()

    def dump(k, _):
        r0 = pl.multiple_of(sid * RT + k * RT, RT)
        for c, t in enumerate(tabs):
            pltpu.sync_copy(t.at[pl.ds(r0, RT)],
                            t_out.at[cid, c, pl.ds(r0, RT)])
        return 0
    lax.fori_loop(0, 1, dump, 0)


def _apply_body(img_hbm, seg_hbm, t_in, fvr_hbm, out_hbm, idx2d, xb0, xb1,
                xb2, a0, a1, a2, sb, cb, ab, fb, obuf, aj0, aj1, aj2, sem,
                sem2):
    cid = lax.axis_index("c")
    sid = lax.axis_index("s")
    wid = sid * 2 + cid
    iota = _iota16()
    ajs = (aj0, aj1, aj2)

    def pa(k, _):
        r0 = pl.multiple_of(sid * RT + k * MA, MA)
        pltpu.sync_copy(t_in.at[0, 3, pl.ds(r0, MA)], cb)
        pltpu.sync_copy(t_in.at[1, 3, pl.ds(r0, MA)], ab)

        def inv_cnt(i, _):
            s = pl.ds(i * 16, 16)
            cb[s] = 1.0 / jnp.maximum(cb[s] + ab[s], 1.0)
            return 0
        lax.fori_loop(0, MA // 16, inv_cnt, 0)

        pltpu.sync_copy(fvr_hbm.at[pl.ds(pl.multiple_of(r0 * 4, MA), 4 * MA)],
                        fb)
        for c in range(3):
            pltpu.sync_copy(t_in.at[0, c, pl.ds(r0, MA)], sb)
            pltpu.sync_copy(t_in.at[1, c, pl.ds(r0, MA)], ab)

            def ca(i, _):
                s = pl.ds(i * 16, 16)
                foff = ((i >> 3) * 4 + c) * 128 + (i & 7) * 16
                f = fb[pl.ds(foff, 16)]
                ab[s] = f - (sb[s] + ab[s]) * cb[s]
                return 0
            lax.fori_loop(0, MA // 16, ca, 0)
            pltpu.sync_copy(ab, ajs[c].at[pl.ds(r0, MA)])
        return 0
    lax.fori_loop(0, RT // MA, pa, 0)
    plsc.subcore_barrier()

    base_px = wid * P

    def pb(j, _):
        px0 = base_px + j * M
        lds = [pltpu.async_copy(
            seg_hbm.at[pl.ds(pl.multiple_of(px0 // 128, G), G)], idx2d,
            sem2)]
        for c, xb in enumerate((xb0, xb1, xb2)):
            off = pl.multiple_of(_x_off(wid, c, j), M)
            lds.append(pltpu.async_copy(img_hbm.at[pl.ds(off, M)], xb, sem2))
        for ld in lds:
            ld.wait()

        def gath(g, _):
            s0 = pl.multiple_of(g * 128, 128)
            idx = idx2d.at[g]
            cps = [pltpu.async_copy(t.at[idx], dst.at[pl.ds(s0, 128)], sem)
                   for t, dst in zip(ajs, (a0, a1, a2))]
            for cp in cps:
                cp.wait()
            return 0
        lax.fori_loop(0, G, gath, 0)

        def cbod(i, _):
            g = i >> 3
            rs = (g >> 5) * 32 + (g & 3) * 8 + ((g >> 2) & 7)
            xoff = rs * 128 + (i & 7) * 16
            s = pl.ds(i * 16, 16)
            for c, (xb, a) in enumerate(zip((xb0, xb1, xb2), (a0, a1, a2))):
                ooff = (g * 4 + c) * 128 + (i & 7) * 16
                obuf[pl.ds(ooff, 16)] = xb[pl.ds(xoff, 16)] + a[s]
            return 0
        lax.fori_loop(0, M // 16, cbod, 0)
        pltpu.sync_copy(obuf, out_hbm.at[pl.ds(pl.multiple_of(px0 * 4, M),
                                               4 * M)])
        return 0
    lax.fori_loop(0, NSUB, pb, 0)


def kernel(img, seg, fV_regions):
    img_stripe = (img.reshape(B, C, H // 8, 8, W // 128, 128)
                  .transpose(0, 1, 2, 4, 3, 5).reshape(-1))
    seg2d = seg.reshape(-1, 128)
    fvr_g4 = (jnp.pad(fV_regions, ((0, 0), (0, 1)))
              .reshape(NV // 128, 128, 4).transpose(0, 2, 1)
              .reshape(NV // 128 * 4, 128).reshape(-1))
    mesh = plsc.VectorSubcoreMesh(core_axis_name="c", subcore_axis_name="s")

    accum = pl.kernel(
        _accum_body,
        out_type=jax.ShapeDtypeStruct((2, 4, NV), jnp.float32),
        mesh=mesh,
        compiler_params=pltpu.CompilerParams(needs_layout_passes=False),
        scratch_types=[
            pltpu.VMEM((G, 128), jnp.int32),
            pltpu.VMEM((M,), jnp.float32),
            pltpu.VMEM((M,), jnp.float32),
            pltpu.VMEM((M,), jnp.float32),
            pltpu.VMEM((M,), jnp.float32),
            pltpu.VMEM((M,), jnp.float32),
            pltpu.VMEM_SHARED((NV,), jnp.float32),
            pltpu.VMEM_SHARED((NV,), jnp.float32),
            pltpu.VMEM_SHARED((NV,), jnp.float32),
            pltpu.VMEM_SHARED((NV,), jnp.float32),
            pltpu.SemaphoreType.DMA,
            pltpu.SemaphoreType.DMA,
        ],
    )
    t_part = accum(img_stripe, seg2d)

    apply_k = pl.kernel(
        _apply_body,
        out_type=jax.ShapeDtypeStruct((N * 4,), jnp.float32),
        mesh=mesh,
        compiler_params=pltpu.CompilerParams(needs_layout_passes=False),
        scratch_types=[
            pltpu.VMEM((G, 128), jnp.int32),
            pltpu.VMEM((M,), jnp.float32),
            pltpu.VMEM((M,), jnp.float32),
            pltpu.VMEM((M,), jnp.float32),
            pltpu.VMEM((M,), jnp.float32),
            pltpu.VMEM((M,), jnp.float32),
            pltpu.VMEM((M,), jnp.float32),
            pltpu.VMEM((MA,), jnp.float32),
            pltpu.VMEM((MA,), jnp.float32),
            pltpu.VMEM((MA,), jnp.float32),
            pltpu.VMEM((4 * MA,), jnp.float32),
            pltpu.VMEM((4 * M,), jnp.float32),
            pltpu.VMEM_SHARED((NV,), jnp.float32),
            pltpu.VMEM_SHARED((NV,), jnp.float32),
            pltpu.VMEM_SHARED((NV,), jnp.float32),
            pltpu.SemaphoreType.DMA,
            pltpu.SemaphoreType.DMA,
        ],
    )
    out = apply_k(img_stripe, seg2d, t_part, fvr_g4)
    return (out.reshape(N // 128, 4, 128).transpose(0, 2, 1)
            .reshape(N, 4)[:, :3])


# speedup vs baseline: 117.6244x; 1.1573x over previous
import jax
import jax.numpy as jnp
from jax import lax
from jax.experimental import pallas as pl
from jax.experimental.pallas import tpu as pltpu
from jax.experimental.pallas import tpu_sc as plsc

B, C, H, W = 8, 3, 512, 512
HW = H * W
N = B * HW
NV = 131072
NW = 32
P = N // NW
M = 8192
G = M // 128
NSUB = P // M
RT = NV // 16
MA = 2048


def _iota16():
    return lax.iota(jnp.int32, 16)


def _x_off(wid, c, j):
    b = wid // 4
    ht = (wid % 4) * 16 + j * 2
    return ((b * C + c) * 64 + ht) * 4096


def _accum_body(img_hbm, seg_hbm, t_out, idx2d, xb0, xb1, xb2, ones, zb,
                tb0, tb1, tb2, tb3, sem, sem2):
    cid = lax.axis_index("c")
    sid = lax.axis_index("s")
    wid = sid * 2 + cid
    tabs = (tb0, tb1, tb2, tb3)

    def fill(i, _):
        ones[pl.ds(i * 16, 16)] = jnp.full((16,), 1.0, jnp.float32)
        zb[pl.ds(i * 16, 16)] = jnp.zeros((16,), jnp.float32)
        return 0
    lax.fori_loop(0, M // 16, fill, 0)

    def ztab(k, _):
        r0 = pl.multiple_of(sid * RT + k * 2048, 2048)
        for t in tabs:
            pltpu.sync_copy(zb.at[pl.ds(0, 2048)], t.at[pl.ds(r0, 2048)])
        return 0
    lax.fori_loop(0, RT // 2048, ztab, 0)
    plsc.subcore_barrier()

    base_px = wid * P

    def sub(j, _):
        px0 = base_px + j * M
        lds = [pltpu.async_copy(
            seg_hbm.at[pl.ds(pl.multiple_of(px0 // 128, G), G)], idx2d,
            sem2)]
        for c, xb in enumerate((xb0, xb1, xb2)):
            off = pl.multiple_of(_x_off(wid, c, j), M)
            lds.append(pltpu.async_copy(img_hbm.at[pl.ds(off, M)], xb, sem2))
        for ld in lds:
            ld.wait()

        def scat(q, _):
            cps = []
            for u in range(2):
                g = q * 2 + u
                rs = (g >> 5) * 32 + (g & 3) * 8 + ((g >> 2) & 7)
                s0 = pl.multiple_of(rs * 128, 128)
                idx = idx2d.at[g]
                cps += [pltpu.async_copy(src.at[pl.ds(s0, 128)], t.at[idx],
                                         sem, add=True)
                        for src, t in zip((xb0, xb1, xb2), tabs)]
                cps.append(pltpu.async_copy(ones.at[pl.ds(0, 128)],
                                            tabs[3].at[idx], sem, add=True))
            for cp in cps:
                cp.wait()
            return 0
        lax.fori_loop(0, G // 2, scat, 0)
        return 0
    lax.fori_loop(0, NSUB, sub, 0)
    plsc.subcore_barrier()

    def dump(k, _):
        r0 = pl.multiple_of(sid * RT + k * RT, RT)
        for c, t in enumerate(tabs):
            pltpu.sync_copy(t.at[pl.ds(r0, RT)],
                            t_out.at[cid, c, pl.ds(r0, RT)])
        return 0
    lax.fori_loop(0, 1, dump, 0)


def _apply_body(img_hbm, seg_hbm, t_in, fvr_hbm, out_hbm, idx2d, xb0, xb1,
                xb2, a0, a1, a2, sb, cb, ab, fb, obuf, aj0, aj1, aj2, sem,
                sem2):
    cid = lax.axis_index("c")
    sid = lax.axis_index("s")
    wid = sid * 2 + cid
    iota = _iota16()
    ajs = (aj0, aj1, aj2)

    def pa(k, _):
        r0 = pl.multiple_of(sid * RT + k * MA, MA)
        pltpu.sync_copy(t_in.at[0, 3, pl.ds(r0, MA)], cb)
        pltpu.sync_copy(t_in.at[1, 3, pl.ds(r0, MA)], ab)

        def inv_cnt(i, _):
            s = pl.ds(i * 16, 16)
            cb[s] = 1.0 / jnp.maximum(cb[s] + ab[s], 1.0)
            return 0
        lax.fori_loop(0, MA // 16, inv_cnt, 0)

        pltpu.sync_copy(fvr_hbm.at[pl.ds(pl.multiple_of(r0 * 4, MA), 4 * MA)],
                        fb)
        for c in range(3):
            pltpu.sync_copy(t_in.at[0, c, pl.ds(r0, MA)], sb)
            pltpu.sync_copy(t_in.at[1, c, pl.ds(r0, MA)], ab)

            def ca(i, _):
                s = pl.ds(i * 16, 16)
                foff = ((i >> 3) * 4 + c) * 128 + (i & 7) * 16
                f = fb[pl.ds(foff, 16)]
                ab[s] = f - (sb[s] + ab[s]) * cb[s]
                return 0
            lax.fori_loop(0, MA // 16, ca, 0)
            pltpu.sync_copy(ab, ajs[c].at[pl.ds(r0, MA)])
        return 0
    lax.fori_loop(0, RT // MA, pa, 0)
    plsc.subcore_barrier()

    base_px = wid * P

    def pb(j, _):
        px0 = base_px + j * M
        lds = [pltpu.async_copy(
            seg_hbm.at[pl.ds(pl.multiple_of(px0 // 128, G), G)], idx2d,
            sem2)]
        for c, xb in enumerate((xb0, xb1, xb2)):
            off = pl.multiple_of(_x_off(wid, c, j), M)
            lds.append(pltpu.async_copy(img_hbm.at[pl.ds(off, M)], xb, sem2))
        for ld in lds:
            ld.wait()

        def gath(q, _):
            cps = []
            for u in range(4):
                g = q * 4 + u
                s0 = pl.multiple_of(g * 128, 128)
                idx = idx2d.at[g]
                cps += [pltpu.async_copy(t.at[idx], dst.at[pl.ds(s0, 128)],
                                         sem)
                        for t, dst in zip(ajs, (a0, a1, a2))]
            for cp in cps:
                cp.wait()
            return 0
        lax.fori_loop(0, G // 4, gath, 0)

        def cbod(i, _):
            g = i >> 3
            rs = (g >> 5) * 32 + (g & 3) * 8 + ((g >> 2) & 7)
            xoff = rs * 128 + (i & 7) * 16
            s = pl.ds(i * 16, 16)
            for c, (xb, a) in enumerate(zip((xb0, xb1, xb2), (a0, a1, a2))):
                ooff = (g * 4 + c) * 128 + (i & 7) * 16
                obuf[pl.ds(ooff, 16)] = xb[pl.ds(xoff, 16)] + a[s]
            return 0
        lax.fori_loop(0, M // 16, cbod, 0)
        pltpu.sync_copy(obuf, out_hbm.at[pl.ds(pl.multiple_of(px0 * 4, M),
                                               4 * M)])
        return 0
    lax.fori_loop(0, NSUB, pb, 0)


def kernel(img, seg, fV_regions):
    img_stripe = (img.reshape(B, C, H // 8, 8, W // 128, 128)
                  .transpose(0, 1, 2, 4, 3, 5).reshape(-1))
    seg2d = seg.reshape(-1, 128)
    fvr_g4 = (jnp.pad(fV_regions, ((0, 0), (0, 1)))
              .reshape(NV // 128, 128, 4).transpose(0, 2, 1)
              .reshape(NV // 128 * 4, 128).reshape(-1))
    mesh = plsc.VectorSubcoreMesh(core_axis_name="c", subcore_axis_name="s")

    accum = pl.kernel(
        _accum_body,
        out_type=jax.ShapeDtypeStruct((2, 4, NV), jnp.float32),
        mesh=mesh,
        compiler_params=pltpu.CompilerParams(needs_layout_passes=False),
        scratch_types=[
            pltpu.VMEM((G, 128), jnp.int32),
            pltpu.VMEM((M,), jnp.float32),
            pltpu.VMEM((M,), jnp.float32),
            pltpu.VMEM((M,), jnp.float32),
            pltpu.VMEM((M,), jnp.float32),
            pltpu.VMEM((M,), jnp.float32),
            pltpu.VMEM_SHARED((NV,), jnp.float32),
            pltpu.VMEM_SHARED((NV,), jnp.float32),
            pltpu.VMEM_SHARED((NV,), jnp.float32),
            pltpu.VMEM_SHARED((NV,), jnp.float32),
            pltpu.SemaphoreType.DMA,
            pltpu.SemaphoreType.DMA,
        ],
    )
    t_part = accum(img_stripe, seg2d)

    apply_k = pl.kernel(
        _apply_body,
        out_type=jax.ShapeDtypeStruct((N * 4,), jnp.float32),
        mesh=mesh,
        compiler_params=pltpu.CompilerParams(needs_layout_passes=False),
        scratch_types=[
            pltpu.VMEM((G, 128), jnp.int32),
            pltpu.VMEM((M,), jnp.float32),
            pltpu.VMEM((M,), jnp.float32),
            pltpu.VMEM((M,), jnp.float32),
            pltpu.VMEM((M,), jnp.float32),
            pltpu.VMEM((M,), jnp.float32),
            pltpu.VMEM((M,), jnp.float32),
            pltpu.VMEM((MA,), jnp.float32),
            pltpu.VMEM((MA,), jnp.float32),
            pltpu.VMEM((MA,), jnp.float32),
            pltpu.VMEM((4 * MA,), jnp.float32),
            pltpu.VMEM((4 * M,), jnp.float32),
            pltpu.VMEM_SHARED((NV,), jnp.float32),
            pltpu.VMEM_SHARED((NV,), jnp.float32),
            pltpu.VMEM_SHARED((NV,), jnp.float32),
            pltpu.SemaphoreType.DMA,
            pltpu.SemaphoreType.DMA,
        ],
    )
    out = apply_k(img_stripe, seg2d, t_part, fvr_g4)
    return (out.reshape(N // 128, 4, 128).transpose(0, 2, 1)
            .reshape(N, 4)[:, :3])


# speedup vs baseline: 122.5895x; 1.0422x over previous
import jax
import jax.numpy as jnp
from jax import lax
from jax.experimental import pallas as pl
from jax.experimental.pallas import tpu as pltpu
from jax.experimental.pallas import tpu_sc as plsc

B, C, H, W = 8, 3, 512, 512
HW = H * W
N = B * HW
NV = 131072
NW = 32
P = N // NW
M = 8192
G = M // 128
NSUB = P // M
RT = NV // 16
MA = 2048


def _iota16():
    return lax.iota(jnp.int32, 16)


def _x_off(wid, c, j):
    b = wid // 4
    ht = (wid % 4) * 16 + j * 2
    return ((b * C + c) * 64 + ht) * 4096


def _accum_body(img_hbm, seg_hbm, t_out, idx2d, xb0, xb1, xb2, ones, zb,
                tb0, tb1, tb2, tb3, sem, sem2):
    cid = lax.axis_index("c")
    sid = lax.axis_index("s")
    wid = sid * 2 + cid
    tabs = (tb0, tb1, tb2, tb3)

    def fill(i, _):
        ones[pl.ds(i * 16, 16)] = jnp.full((16,), 1.0, jnp.float32)
        zb[pl.ds(i * 16, 16)] = jnp.zeros((16,), jnp.float32)
        return 0
    lax.fori_loop(0, M // 16, fill, 0)

    def ztab(k, _):
        r0 = pl.multiple_of(sid * RT + k * 2048, 2048)
        for t in tabs:
            pltpu.sync_copy(zb.at[pl.ds(0, 2048)], t.at[pl.ds(r0, 2048)])
        return 0
    lax.fori_loop(0, RT // 2048, ztab, 0)
    plsc.subcore_barrier()

    base_px = wid * P

    def sub(j, _):
        px0 = base_px + j * M
        lds = [pltpu.async_copy(
            seg_hbm.at[pl.ds(pl.multiple_of(px0 // 128, G), G)], idx2d,
            sem2)]
        for c, xb in enumerate((xb0, xb1, xb2)):
            off = pl.multiple_of(_x_off(wid, c, j), M)
            lds.append(pltpu.async_copy(img_hbm.at[pl.ds(off, M)], xb, sem2))
        for ld in lds:
            ld.wait()

        def scat(q, _):
            cps = []
            for u in range(4):
                g = q * 4 + u
                rs = (g >> 5) * 32 + (g & 3) * 8 + ((g >> 2) & 7)
                s0 = pl.multiple_of(rs * 128, 128)
                idx = idx2d.at[g]
                cps += [pltpu.async_copy(src.at[pl.ds(s0, 128)], t.at[idx],
                                         sem, add=True)
                        for src, t in zip((xb0, xb1, xb2), tabs)]
                cps.append(pltpu.async_copy(ones.at[pl.ds(0, 128)],
                                            tabs[3].at[idx], sem, add=True))
            for cp in cps:
                cp.wait()
            return 0
        lax.fori_loop(0, G // 4, scat, 0)
        return 0
    lax.fori_loop(0, NSUB, sub, 0)
    plsc.subcore_barrier()

    def dump(k, _):
        r0 = pl.multiple_of(sid * RT + k * RT, RT)
        for c, t in enumerate(tabs):
            pltpu.sync_copy(t.at[pl.ds(r0, RT)],
                            t_out.at[cid, c, pl.ds(r0, RT)])
        return 0
    lax.fori_loop(0, 1, dump, 0)


def _apply_body(img_hbm, seg_hbm, t_in, fvr_hbm, out_hbm, idx2d, xb0, xb1,
                xb2, a0, a1, a2, sb, cb, ab, fb, obuf, aj0, aj1, aj2, sem,
                sem2):
    cid = lax.axis_index("c")
    sid = lax.axis_index("s")
    wid = sid * 2 + cid
    iota = _iota16()
    ajs = (aj0, aj1, aj2)

    def pa(k, _):
        r0 = pl.multiple_of(sid * RT + k * MA, MA)
        pltpu.sync_copy(t_in.at[0, 3, pl.ds(r0, MA)], cb)
        pltpu.sync_copy(t_in.at[1, 3, pl.ds(r0, MA)], ab)

        def inv_cnt(i, _):
            s = pl.ds(i * 16, 16)
            cb[s] = 1.0 / jnp.maximum(cb[s] + ab[s], 1.0)
            return 0
        lax.fori_loop(0, MA // 16, inv_cnt, 0)

        pltpu.sync_copy(fvr_hbm.at[pl.ds(pl.multiple_of(r0 * 4, MA), 4 * MA)],
                        fb)
        for c in range(3):
            pltpu.sync_copy(t_in.at[0, c, pl.ds(r0, MA)], sb)
            pltpu.sync_copy(t_in.at[1, c, pl.ds(r0, MA)], ab)

            def ca(i, _):
                s = pl.ds(i * 16, 16)
                foff = ((i >> 3) * 4 + c) * 128 + (i & 7) * 16
                f = fb[pl.ds(foff, 16)]
                ab[s] = f - (sb[s] + ab[s]) * cb[s]
                return 0
            lax.fori_loop(0, MA // 16, ca, 0)
            pltpu.sync_copy(ab, ajs[c].at[pl.ds(r0, MA)])
        return 0
    lax.fori_loop(0, RT // MA, pa, 0)
    plsc.subcore_barrier()

    base_px = wid * P

    def pb(j, _):
        px0 = base_px + j * M
        lds = [pltpu.async_copy(
            seg_hbm.at[pl.ds(pl.multiple_of(px0 // 128, G), G)], idx2d,
            sem2)]
        for c, xb in enumerate((xb0, xb1, xb2)):
            off = pl.multiple_of(_x_off(wid, c, j), M)
            lds.append(pltpu.async_copy(img_hbm.at[pl.ds(off, M)], xb, sem2))
        for ld in lds:
            ld.wait()

        def gath(q, _):
            cps = []
            for u in range(8):
                g = q * 8 + u
                s0 = pl.multiple_of(g * 128, 128)
                idx = idx2d.at[g]
                cps += [pltpu.async_copy(t.at[idx], dst.at[pl.ds(s0, 128)],
                                         sem)
                        for t, dst in zip(ajs, (a0, a1, a2))]
            for cp in cps:
                cp.wait()
            return 0
        lax.fori_loop(0, G // 8, gath, 0)

        def cbod(i, _):
            g = i >> 3
            rs = (g >> 5) * 32 + (g & 3) * 8 + ((g >> 2) & 7)
            xoff = rs * 128 + (i & 7) * 16
            s = pl.ds(i * 16, 16)
            for c, (xb, a) in enumerate(zip((xb0, xb1, xb2), (a0, a1, a2))):
                ooff = (g * 4 + c) * 128 + (i & 7) * 16
                obuf[pl.ds(ooff, 16)] = xb[pl.ds(xoff, 16)] + a[s]
            return 0
        lax.fori_loop(0, M // 16, cbod, 0)
        pltpu.sync_copy(obuf, out_hbm.at[pl.ds(pl.multiple_of(px0 * 4, M),
                                               4 * M)])
        return 0
    lax.fori_loop(0, NSUB, pb, 0)


def kernel(img, seg, fV_regions):
    img_stripe = (img.reshape(B, C, H // 8, 8, W // 128, 128)
                  .transpose(0, 1, 2, 4, 3, 5).reshape(-1))
    seg2d = seg.reshape(-1, 128)
    fvr_g4 = (jnp.pad(fV_regions, ((0, 0), (0, 1)))
              .reshape(NV // 128, 128, 4).transpose(0, 2, 1)
              .reshape(NV // 128 * 4, 128).reshape(-1))
    mesh = plsc.VectorSubcoreMesh(core_axis_name="c", subcore_axis_name="s")

    accum = pl.kernel(
        _accum_body,
        out_type=jax.ShapeDtypeStruct((2, 4, NV), jnp.float32),
        mesh=mesh,
        compiler_params=pltpu.CompilerParams(needs_layout_passes=False),
        scratch_types=[
            pltpu.VMEM((G, 128), jnp.int32),
            pltpu.VMEM((M,), jnp.float32),
            pltpu.VMEM((M,), jnp.float32),
            pltpu.VMEM((M,), jnp.float32),
            pltpu.VMEM((M,), jnp.float32),
            pltpu.VMEM((M,), jnp.float32),
            pltpu.VMEM_SHARED((NV,), jnp.float32),
            pltpu.VMEM_SHARED((NV,), jnp.float32),
            pltpu.VMEM_SHARED((NV,), jnp.float32),
            pltpu.VMEM_SHARED((NV,), jnp.float32),
            pltpu.SemaphoreType.DMA,
            pltpu.SemaphoreType.DMA,
        ],
    )
    t_part = accum(img_stripe, seg2d)

    apply_k = pl.kernel(
        _apply_body,
        out_type=jax.ShapeDtypeStruct((N * 4,), jnp.float32),
        mesh=mesh,
        compiler_params=pltpu.CompilerParams(needs_layout_passes=False),
        scratch_types=[
            pltpu.VMEM((G, 128), jnp.int32),
            pltpu.VMEM((M,), jnp.float32),
            pltpu.VMEM((M,), jnp.float32),
            pltpu.VMEM((M,), jnp.float32),
            pltpu.VMEM((M,), jnp.float32),
            pltpu.VMEM((M,), jnp.float32),
            pltpu.VMEM((M,), jnp.float32),
            pltpu.VMEM((MA,), jnp.float32),
            pltpu.VMEM((MA,), jnp.float32),
            pltpu.VMEM((MA,), jnp.float32),
            pltpu.VMEM((4 * MA,), jnp.float32),
            pltpu.VMEM((4 * M,), jnp.float32),
            pltpu.VMEM_SHARED((NV,), jnp.float32),
            pltpu.VMEM_SHARED((NV,), jnp.float32),
            pltpu.VMEM_SHARED((NV,), jnp.float32),
            pltpu.SemaphoreType.DMA,
            pltpu.SemaphoreType.DMA,
        ],
    )
    out = apply_k(img_stripe, seg2d, t_part, fvr_g4)
    return (out.reshape(N // 128, 4, 128).transpose(0, 2, 1)
            .reshape(N, 4)[:, :3])


# speedup vs baseline: 132.3115x; 1.0793x over previous
import jax
import jax.numpy as jnp
from jax import lax
from jax.experimental import pallas as pl
from jax.experimental.pallas import tpu as pltpu
from jax.experimental.pallas import tpu_sc as plsc

B, C, H, W = 8, 3, 512, 512
HW = H * W
N = B * HW
NV = 131072
NW = 32
P = N // NW
M = 8192
G = M // 128
NSUB = P // M
RT = NV // 16
MA = 2048


def _iota16():
    return lax.iota(jnp.int32, 16)


def _x_off(wid, c, j):
    b = wid // 4
    ht = (wid % 4) * 16 + j * 2
    return ((b * C + c) * 64 + ht) * 4096


def _accum_body(img_hbm, seg_hbm, t_out, idx2d, xb0, xb1, xb2, ones, zb,
                tb0, tb1, tb2, tb3, sem, sem2):
    cid = lax.axis_index("c")
    sid = lax.axis_index("s")
    wid = sid * 2 + cid
    tabs = (tb0, tb1, tb2, tb3)

    def fill(i, _):
        ones[pl.ds(i * 16, 16)] = jnp.full((16,), 1.0, jnp.float32)
        zb[pl.ds(i * 16, 16)] = jnp.zeros((16,), jnp.float32)
        return 0
    lax.fori_loop(0, M // 16, fill, 0)

    def ztab(k, _):
        r0 = pl.multiple_of(sid * RT + k * 2048, 2048)
        for t in tabs:
            pltpu.sync_copy(zb.at[pl.ds(0, 2048)], t.at[pl.ds(r0, 2048)])
        return 0
    lax.fori_loop(0, RT // 2048, ztab, 0)
    plsc.subcore_barrier()

    base_px = wid * P

    def sub(j, _):
        px0 = base_px + j * M
        lds = [pltpu.async_copy(
            seg_hbm.at[pl.ds(pl.multiple_of(px0 // 128, G), G)], idx2d,
            sem2)]
        for c, xb in enumerate((xb0, xb1, xb2)):
            off = pl.multiple_of(_x_off(wid, c, j), M)
            lds.append(pltpu.async_copy(img_hbm.at[pl.ds(off, M)], xb, sem2))
        for ld in lds:
            ld.wait()

        def scat(q, _):
            cps = []
            for u in range(4):
                g = q * 4 + u
                rs = (g >> 5) * 32 + (g & 3) * 8 + ((g >> 2) & 7)
                s0 = pl.multiple_of(rs * 128, 128)
                idx = idx2d.at[g]
                cps += [pltpu.async_copy(src.at[pl.ds(s0, 128)], t.at[idx],
                                         sem, add=True)
                        for src, t in zip((xb0, xb1, xb2), tabs)]
                cps.append(pltpu.async_copy(ones.at[pl.ds(0, 128)],
                                            tabs[3].at[idx], sem, add=True))
            for cp in cps:
                cp.wait()
            return 0
        lax.fori_loop(0, G // 4, scat, 0)
        return 0
    lax.fori_loop(0, NSUB, sub, 0)
    plsc.subcore_barrier()

    def dump(k, _):
        r0 = pl.multiple_of(sid * RT + k * RT, RT)
        for c, t in enumerate(tabs):
            pltpu.sync_copy(t.at[pl.ds(r0, RT)],
                            t_out.at[cid, c, pl.ds(r0, RT)])
        return 0
    lax.fori_loop(0, 1, dump, 0)


def _apply_body(img_hbm, seg_hbm, t_in, fvr_hbm, out_hbm, idx2d, xb0, xb1,
                xb2, ai, a2b, c0b, sb, cb, ab, fb, pki, obuf, ajp, aj2, sem,
                sem2):
    cid = lax.axis_index("c")
    sid = lax.axis_index("s")
    wid = sid * 2 + cid

    def pa(k, _):
        r0 = pl.multiple_of(sid * RT + k * MA, MA)
        pltpu.sync_copy(t_in.at[0, 3, pl.ds(r0, MA)], cb)
        pltpu.sync_copy(t_in.at[1, 3, pl.ds(r0, MA)], ab)

        def inv_cnt(i, _):
            s = pl.ds(i * 16, 16)
            cb[s] = 1.0 / jnp.maximum(cb[s] + ab[s], 1.0)
            return 0
        lax.fori_loop(0, MA // 16, inv_cnt, 0)

        pltpu.sync_copy(fvr_hbm.at[pl.ds(pl.multiple_of(r0 * 4, MA), 4 * MA)],
                        fb)
        pltpu.sync_copy(t_in.at[0, 0, pl.ds(r0, MA)], sb)
        pltpu.sync_copy(t_in.at[1, 0, pl.ds(r0, MA)], ab)

        def ca0(i, _):
            s = pl.ds(i * 16, 16)
            foff = ((i >> 3) * 4) * 128 + (i & 7) * 16
            f = fb[pl.ds(foff, 16)]
            c0b[s] = f - (sb[s] + ab[s]) * cb[s]
            return 0
        lax.fori_loop(0, MA // 16, ca0, 0)

        pltpu.sync_copy(t_in.at[0, 1, pl.ds(r0, MA)], sb)
        pltpu.sync_copy(t_in.at[1, 1, pl.ds(r0, MA)], ab)

        def ca1(i, _):
            s = pl.ds(i * 16, 16)
            foff = ((i >> 3) * 4 + 1) * 128 + (i & 7) * 16
            f = fb[pl.ds(foff, 16)]
            v1 = f - (sb[s] + ab[s]) * cb[s]
            pki[s] = plsc.bitcast(
                plsc.pack(c0b[s], v1, format=plsc.PackFormat.INTERLEAVED),
                jnp.int32)
            return 0
        lax.fori_loop(0, MA // 16, ca1, 0)
        pltpu.sync_copy(pki, ajp.at[pl.ds(r0, MA)])

        pltpu.sync_copy(t_in.at[0, 2, pl.ds(r0, MA)], sb)
        pltpu.sync_copy(t_in.at[1, 2, pl.ds(r0, MA)], ab)

        def ca2(i, _):
            s = pl.ds(i * 16, 16)
            foff = ((i >> 3) * 4 + 2) * 128 + (i & 7) * 16
            f = fb[pl.ds(foff, 16)]
            ab[s] = f - (sb[s] + ab[s]) * cb[s]
            return 0
        lax.fori_loop(0, MA // 16, ca2, 0)
        pltpu.sync_copy(ab, aj2.at[pl.ds(r0, MA)])
        return 0
    lax.fori_loop(0, RT // MA, pa, 0)
    plsc.subcore_barrier()

    base_px = wid * P

    def pb(j, _):
        px0 = base_px + j * M
        lds = [pltpu.async_copy(
            seg_hbm.at[pl.ds(pl.multiple_of(px0 // 128, G), G)], idx2d,
            sem2)]
        for c, xb in enumerate((xb0, xb1, xb2)):
            off = pl.multiple_of(_x_off(wid, c, j), M)
            lds.append(pltpu.async_copy(img_hbm.at[pl.ds(off, M)], xb, sem2))
        for ld in lds:
            ld.wait()

        def gath(q, _):
            cps = []
            for u in range(8):
                g = q * 8 + u
                s0 = pl.multiple_of(g * 128, 128)
                idx = idx2d.at[g]
                cps.append(pltpu.async_copy(ajp.at[idx],
                                            ai.at[pl.ds(s0, 128)], sem))
                cps.append(pltpu.async_copy(aj2.at[idx],
                                            a2b.at[pl.ds(s0, 128)], sem))
            for cp in cps:
                cp.wait()
            return 0
        lax.fori_loop(0, G // 8, gath, 0)

        def cbod(i, _):
            g = i >> 3
            rs = (g >> 5) * 32 + (g & 3) * 8 + ((g >> 2) & 7)
            xoff = rs * 128 + (i & 7) * 16
            s = pl.ds(i * 16, 16)
            a0v, a1v = plsc.unpack(
                plsc.bitcast(ai[s], jnp.bfloat16),
                format=plsc.PackFormat.INTERLEAVED,
                preferred_element_type=jnp.float32)
            for c, (xb, a) in enumerate(zip((xb0, xb1, xb2),
                                            (a0v, a1v, None))):
                ooff = (g * 4 + c) * 128 + (i & 7) * 16
                av = a2b[s] if c == 2 else a
                obuf[pl.ds(ooff, 16)] = xb[pl.ds(xoff, 16)] + av
            return 0
        lax.fori_loop(0, M // 16, cbod, 0)
        pltpu.sync_copy(obuf, out_hbm.at[pl.ds(pl.multiple_of(px0 * 4, M),
                                               4 * M)])
        return 0
    lax.fori_loop(0, NSUB, pb, 0)


def kernel(img, seg, fV_regions):
    img_stripe = (img.reshape(B, C, H // 8, 8, W // 128, 128)
                  .transpose(0, 1, 2, 4, 3, 5).reshape(-1))
    seg2d = seg.reshape(-1, 128)
    fvr_g4 = (jnp.pad(fV_regions, ((0, 0), (0, 1)))
              .reshape(NV // 128, 128, 4).transpose(0, 2, 1)
              .reshape(NV // 128 * 4, 128).reshape(-1))
    mesh = plsc.VectorSubcoreMesh(core_axis_name="c", subcore_axis_name="s")

    accum = pl.kernel(
        _accum_body,
        out_type=jax.ShapeDtypeStruct((2, 4, NV), jnp.float32),
        mesh=mesh,
        compiler_params=pltpu.CompilerParams(needs_layout_passes=False),
        scratch_types=[
            pltpu.VMEM((G, 128), jnp.int32),
            pltpu.VMEM((M,), jnp.float32),
            pltpu.VMEM((M,), jnp.float32),
            pltpu.VMEM((M,), jnp.float32),
            pltpu.VMEM((M,), jnp.float32),
            pltpu.VMEM((M,), jnp.float32),
            pltpu.VMEM_SHARED((NV,), jnp.float32),
            pltpu.VMEM_SHARED((NV,), jnp.float32),
            pltpu.VMEM_SHARED((NV,), jnp.float32),
            pltpu.VMEM_SHARED((NV,), jnp.float32),
            pltpu.SemaphoreType.DMA,
            pltpu.SemaphoreType.DMA,
        ],
    )
    t_part = accum(img_stripe, seg2d)

    apply_k = pl.kernel(
        _apply_body,
        out_type=jax.ShapeDtypeStruct((N * 4,), jnp.float32),
        mesh=mesh,
        compiler_params=pltpu.CompilerParams(needs_layout_passes=False),
        scratch_types=[
            pltpu.VMEM((G, 128), jnp.int32),
            pltpu.VMEM((M,), jnp.float32),
            pltpu.VMEM((M,), jnp.float32),
            pltpu.VMEM((M,), jnp.float32),
            pltpu.VMEM((M,), jnp.int32),
            pltpu.VMEM((M,), jnp.float32),
            pltpu.VMEM((MA,), jnp.float32),
            pltpu.VMEM((MA,), jnp.float32),
            pltpu.VMEM((MA,), jnp.float32),
            pltpu.VMEM((MA,), jnp.float32),
            pltpu.VMEM((4 * MA,), jnp.float32),
            pltpu.VMEM((MA,), jnp.int32),
            pltpu.VMEM((4 * M,), jnp.float32),
            pltpu.VMEM_SHARED((NV,), jnp.int32),
            pltpu.VMEM_SHARED((NV,), jnp.float32),
            pltpu.SemaphoreType.DMA,
            pltpu.SemaphoreType.DMA,
        ],
    )
    out = apply_k(img_stripe, seg2d, t_part, fvr_g4)
    return (out.reshape(N // 128, 4, 128).transpose(0, 2, 1)
            .reshape(N, 4)[:, :3])
